# baseline (speedup 1.0000x reference)
import functools

import jax
import jax.numpy as jnp
from jax import lax
from jax.experimental import pallas as pl
from jax.experimental.pallas import tpu as pltpu
from jax.experimental.pallas import tpu_sc as plsc

VOCAB = 23236
VOCABP = 23240
EMBED_DIM = 1728
SEG = 14
PADDED = SEG * 128
BATCH = 1024
SEQ = 50
SEQP = 56
TAB_ROWS = (VOCABP // 8) * SEG * 8
OUT_ROWS = BATCH * (SEQP // 8) * SEG * 8

NC = 2
NS = 16
NW = NC * NS
RPW = OUT_ROWS // NW
CH = 224
NCHUNK = RPW // CH
SUB = 2
NIDX = CH // SUB

_MESH = plsc.VectorSubcoreMesh(core_axis_name="c", subcore_axis_name="s")


@functools.partial(
    pl.kernel,
    out_type=jax.ShapeDtypeStruct((OUT_ROWS, 128), jnp.float32),
    mesh=_MESH,
    compiler_params=pltpu.CompilerParams(use_tc_tiling_on_sc=False),
    scratch_types=[
        pltpu.VMEM((RPW,), jnp.int32),
        pltpu.VMEM((2, CH, 128), jnp.float32),
        pltpu.SemaphoreType.DMA,
        pltpu.SemaphoreType.DMA,
        pltpu.SemaphoreType.DMA,
    ],
)
def _glyph_gather(idx_hbm, tab_hbm, out_hbm, idx_v, rows_v, gsem, osem0, osem1):
    wid = lax.axis_index("s") * NC + lax.axis_index("c")
    base = wid * RPW
    osems = (osem0, osem1)

    pltpu.sync_copy(idx_hbm.at[pl.ds(wid * RPW, RPW)], idx_v)

    def start_gathers(j, b):
        for q in range(SUB):
            pltpu.async_copy(
                tab_hbm.at[idx_v.at[pl.ds(j * CH + q * NIDX, NIDX)]],
                rows_v.at[b, pl.ds(q * NIDX, NIDX)],
                gsem,
            )

    def wait_gathers(b):
        pltpu.make_async_copy(
            tab_hbm.at[pl.ds(0, CH)], rows_v.at[b], gsem
        ).wait()

    start_gathers(0, 0)

    def pair(p, carry):
        for b in range(2):
            j = 2 * p + b
            wait_gathers(b)

            @pl.when(j >= 1)
            def _():
                pltpu.make_async_copy(
                    rows_v.at[1 - b], out_hbm.at[pl.ds(base, CH)], osems[1 - b]
                ).wait()

            @pl.when(j + 1 < NCHUNK)
            def _():
                start_gathers(j + 1, 1 - b)

            pltpu.async_copy(
                rows_v.at[b], out_hbm.at[pl.ds(base + j * CH, CH)], osems[b]
            )
        return carry

    lax.fori_loop(0, NCHUNK // 2, pair, 0)
    pltpu.make_async_copy(
        rows_v.at[1], out_hbm.at[pl.ds(base, CH)], osem1
    ).wait()


def _pad_body(t_ref, o_ref):
    x = t_ref[...]
    slabs = [x[:, c * 128:(c + 1) * 128] for c in range(SEG - 1)]
    tail = jnp.pad(x[:, (SEG - 1) * 128:], ((0, 0), (0, PADDED - EMBED_DIM)))
    o_ref[...] = jnp.concatenate(slabs + [tail], axis=0)


def _tc_pad(table):
    return pl.pallas_call(
        _pad_body,
        grid=(VOCABP // 8,),
        in_specs=[pl.BlockSpec((8, EMBED_DIM), lambda i: (i, 0))],
        out_specs=pl.BlockSpec((SEG * 8, 128), lambda i: (i, 0)),
        out_shape=jax.ShapeDtypeStruct((TAB_ROWS, 128), jnp.float32),
    )(table)


def _det_body(x_ref, o_ref):
    x = x_ref[0, 0]
    slabs = [x[c] for c in range(SEG - 1)]
    slabs.append(x[SEG - 1, :, : EMBED_DIM - (SEG - 1) * 128])
    o_ref[0] = jnp.concatenate(slabs, axis=1)


def _tc_detile(x5):
    return pl.pallas_call(
        _det_body,
        grid=(BATCH, SEQP // 8),
        in_specs=[pl.BlockSpec((1, 1, SEG, 8, 128), lambda b, r: (b, r, 0, 0, 0))],
        out_specs=pl.BlockSpec((1, 8, EMBED_DIM), lambda b, r: (b, r, 0)),
        out_shape=jax.ShapeDtypeStruct((BATCH, SEQ, EMBED_DIM), jnp.float32),
    )(x5)


def kernel(input_ids, embedding_table):
    ids = input_ids.astype(jnp.int32)
    idsp = jnp.pad(ids, ((0, 0), (0, SEQP - SEQ)))
    vv = idsp.reshape(BATCH, SEQP // 8, 1, 8)
    segs = (
        (vv // 8) * (SEG * 8)
        + (vv % 8)
        + jnp.arange(SEG, dtype=jnp.int32).reshape(1, 1, SEG, 1) * 8
    ).reshape(-1)
    table_seg = _tc_pad(embedding_table)
    out = _glyph_gather(segs, table_seg)
    x5 = out.reshape(BATCH, SEQP // 8, SEG, 8, 128)
    return _tc_detile(x5)


# speedup vs baseline: 3.8095x; 3.8095x over previous
import functools

import jax
import jax.numpy as jnp
from jax import lax
from jax.experimental import pallas as pl
from jax.experimental.pallas import tpu as pltpu
from jax.experimental.pallas import tpu_sc as plsc

VOCAB = 23236
VOCABP = 23240
EMBED_DIM = 1728
SEG = 14
PADDED = SEG * 128
BATCH = 1024
SEQ = 50
SEQP = 56
TAB_ROWS = (VOCABP // 8) * SEG * 8
OUT_ROWS = BATCH * (SEQP // 8) * SEG * 8

NC = 2
NS = 16
NW = NC * NS
RPW = OUT_ROWS // NW
CH = 224
NCHUNK = RPW // CH
SUB = 2
NIDX = CH // SUB

_MESH = plsc.VectorSubcoreMesh(core_axis_name="c", subcore_axis_name="s")


@functools.partial(
    pl.kernel,
    out_type=jax.ShapeDtypeStruct((OUT_ROWS, 128), jnp.float32),
    mesh=_MESH,
    compiler_params=pltpu.CompilerParams(use_tc_tiling_on_sc=False),
    scratch_types=[
        pltpu.VMEM((RPW,), jnp.int32),
        pltpu.VMEM((2, CH, 128), jnp.float32),
        pltpu.SemaphoreType.DMA,
        pltpu.SemaphoreType.DMA,
        pltpu.SemaphoreType.DMA,
    ],
)
def _glyph_gather(idx_hbm, tab_hbm, out_hbm, idx_v, rows_v, gsem, osem0, osem1):
    wid = lax.axis_index("s") * NC + lax.axis_index("c")
    base = wid * RPW
    osems = (osem0, osem1)

    pltpu.sync_copy(idx_hbm.at[pl.ds(wid * RPW, RPW)], idx_v)

    def start_gathers(j, b):
        for q in range(SUB):
            pltpu.async_copy(
                tab_hbm.at[idx_v.at[pl.ds(j * CH + q * NIDX, NIDX)]],
                rows_v.at[b, pl.ds(q * NIDX, NIDX)],
                gsem,
            )

    def wait_gathers(b):
        pltpu.make_async_copy(
            tab_hbm.at[pl.ds(0, CH)], rows_v.at[b], gsem
        ).wait()

    start_gathers(0, 0)

    def pair(p, carry):
        for b in range(2):
            j = 2 * p + b
            wait_gathers(b)

            @pl.when(j >= 1)
            def _():
                pltpu.make_async_copy(
                    rows_v.at[1 - b], out_hbm.at[pl.ds(base, CH)], osems[1 - b]
                ).wait()

            @pl.when(j + 1 < NCHUNK)
            def _():
                start_gathers(j + 1, 1 - b)

            pltpu.async_copy(
                rows_v.at[b], out_hbm.at[pl.ds(base + j * CH, CH)], osems[b]
            )
        return carry

    lax.fori_loop(0, NCHUNK // 2, pair, 0)
    pltpu.make_async_copy(
        rows_v.at[1], out_hbm.at[pl.ds(base, CH)], osem1
    ).wait()


_PAD_RB = 512


def _pad_body(t_ref, o_ref):
    x = t_ref[...]
    tiles = []
    for r in range(_PAD_RB // 8):
        xr = x[r * 8:(r + 1) * 8]
        slabs = [xr[:, c * 128:(c + 1) * 128] for c in range(SEG - 1)]
        slabs.append(
            jnp.pad(xr[:, (SEG - 1) * 128:], ((0, 0), (0, PADDED - EMBED_DIM)))
        )
        tiles.extend(slabs)
    o_ref[...] = jnp.concatenate(tiles, axis=0)


def _tc_pad(table):
    grid = (VOCABP + _PAD_RB - 1) // _PAD_RB
    return pl.pallas_call(
        _pad_body,
        grid=(grid,),
        in_specs=[pl.BlockSpec((_PAD_RB, EMBED_DIM), lambda i: (i, 0))],
        out_specs=pl.BlockSpec((_PAD_RB // 8 * SEG * 8, 128), lambda i: (i, 0)),
        out_shape=jax.ShapeDtypeStruct((TAB_ROWS, 128), jnp.float32),
    )(table)


_DET_BB = 8


def _det_body(x_ref, o_ref):
    x = x_ref[...]
    rows = []
    for r in range(SEQP // 8):
        slabs = [x[:, r, c] for c in range(SEG - 1)]
        slabs.append(x[:, r, SEG - 1, :, : EMBED_DIM - (SEG - 1) * 128])
        rows.append(jnp.concatenate(slabs, axis=2))
    y = jnp.concatenate(rows, axis=1)
    o_ref[...] = y[:, :SEQ]


def _tc_detile(x5):
    return pl.pallas_call(
        _det_body,
        grid=(BATCH // _DET_BB,),
        in_specs=[
            pl.BlockSpec((_DET_BB, SEQP // 8, SEG, 8, 128), lambda b: (b, 0, 0, 0, 0))
        ],
        out_specs=pl.BlockSpec((_DET_BB, SEQ, EMBED_DIM), lambda b: (b, 0, 0)),
        out_shape=jax.ShapeDtypeStruct((BATCH, SEQ, EMBED_DIM), jnp.float32),
    )(x5)


def kernel(input_ids, embedding_table):
    ids = input_ids.astype(jnp.int32)
    idsp = jnp.pad(ids, ((0, 0), (0, SEQP - SEQ)))
    vv = idsp.reshape(BATCH, SEQP // 8, 1, 8)
    segs = (
        (vv // 8) * (SEG * 8)
        + (vv % 8)
        + jnp.arange(SEG, dtype=jnp.int32).reshape(1, 1, SEG, 1) * 8
    ).reshape(-1)
    table_seg = _tc_pad(embedding_table)
    out = _glyph_gather(segs, table_seg)
    x5 = out.reshape(BATCH, SEQP // 8, SEG, 8, 128)
    return _tc_detile(x5)


# speedup vs baseline: 4.9790x; 1.3070x over previous
import functools

import jax
import jax.numpy as jnp
from jax import lax
from jax.experimental import pallas as pl
from jax.experimental.pallas import tpu as pltpu
from jax.experimental.pallas import tpu_sc as plsc

VOCAB = 23236
VOCABP = 23240
EMBED_DIM = 1728
SEG = 14
PADDED = SEG * 128
BATCH = 1024
SEQ = 50
SEQP = 56
TAB_ROWS = VOCAB * SEG
OUT_ROWS = BATCH * SEQ * SEG

NC = 2
NS = 16
NW = NC * NS
RPW = OUT_ROWS // NW
CH = 224
NCHUNK = RPW // CH
SUB = 2
NIDX = CH // SUB

_MESH = plsc.VectorSubcoreMesh(core_axis_name="c", subcore_axis_name="s")


@functools.partial(
    pl.kernel,
    out_type=jax.ShapeDtypeStruct((OUT_ROWS, 128), jnp.float32),
    mesh=_MESH,
    compiler_params=pltpu.CompilerParams(use_tc_tiling_on_sc=False),
    scratch_types=[
        pltpu.VMEM((RPW,), jnp.int32),
        pltpu.VMEM((2, CH, 128), jnp.float32),
        pltpu.SemaphoreType.DMA,
        pltpu.SemaphoreType.DMA,
        pltpu.SemaphoreType.DMA,
    ],
)
def _glyph_gather(idx_hbm, tab_hbm, out_hbm, idx_v, rows_v, gsem, osem0, osem1):
    wid = lax.axis_index("s") * NC + lax.axis_index("c")
    base = wid * RPW
    osems = (osem0, osem1)

    pltpu.sync_copy(idx_hbm.at[pl.ds(wid * RPW, RPW)], idx_v)

    def start_gathers(j, b):
        for q in range(SUB):
            pltpu.async_copy(
                tab_hbm.at[idx_v.at[pl.ds(j * CH + q * NIDX, NIDX)]],
                rows_v.at[b, pl.ds(q * NIDX, NIDX)],
                gsem,
            )

    def wait_gathers(b):
        pltpu.make_async_copy(
            tab_hbm.at[pl.ds(0, CH)], rows_v.at[b], gsem
        ).wait()

    start_gathers(0, 0)

    def pair(p, carry):
        for b in range(2):
            j = 2 * p + b
            wait_gathers(b)

            @pl.when(j >= 1)
            def _():
                pltpu.make_async_copy(
                    rows_v.at[1 - b], out_hbm.at[pl.ds(base, CH)], osems[1 - b]
                ).wait()

            @pl.when(j + 1 < NCHUNK)
            def _():
                start_gathers(j + 1, 1 - b)

            pltpu.async_copy(
                rows_v.at[b], out_hbm.at[pl.ds(base + j * CH, CH)], osems[b]
            )
        return carry

    lax.fori_loop(0, NCHUNK // 2, pair, 0)
    pltpu.make_async_copy(
        rows_v.at[1], out_hbm.at[pl.ds(base, CH)], osem1
    ).wait()


_PAD_RB = 512


def _pad_body(t_ref, o_ref):
    x = t_ref[...]
    y = jnp.pad(x, ((0, 0), (0, PADDED - EMBED_DIM)))
    o_ref[...] = y.reshape(_PAD_RB * SEG, 128)


def _tc_pad(table):
    grid = (VOCAB + _PAD_RB - 1) // _PAD_RB
    return pl.pallas_call(
        _pad_body,
        grid=(grid,),
        in_specs=[pl.BlockSpec((_PAD_RB, EMBED_DIM), lambda i: (i, 0))],
        out_specs=pl.BlockSpec((_PAD_RB * SEG, 128), lambda i: (i, 0)),
        out_shape=jax.ShapeDtypeStruct((TAB_ROWS, 128), jnp.float32),
    )(table)


_DET_BB = 8


def _det_body(x_ref, o_ref):
    x = x_ref[...]
    y = x.reshape(_DET_BB * SEQ, PADDED)[:, :EMBED_DIM]
    o_ref[...] = y.reshape(_DET_BB, SEQ, EMBED_DIM)


def _tc_detile(x):
    return pl.pallas_call(
        _det_body,
        grid=(BATCH // _DET_BB,),
        in_specs=[pl.BlockSpec((_DET_BB * SEQ * SEG, 128), lambda b: (b, 0))],
        out_specs=pl.BlockSpec((_DET_BB, SEQ, EMBED_DIM), lambda b: (b, 0, 0)),
        out_shape=jax.ShapeDtypeStruct((BATCH, SEQ, EMBED_DIM), jnp.float32),
    )(x)


def kernel(input_ids, embedding_table):
    ids = input_ids.reshape(-1).astype(jnp.int32)
    segs = (ids[:, None] * SEG + jnp.arange(SEG, dtype=jnp.int32)).reshape(-1)
    table_seg = _tc_pad(embedding_table)
    out = _glyph_gather(segs, table_seg)
    return _tc_detile(out)


# speedup vs baseline: 5.0608x; 1.0164x over previous
import functools

import jax
import jax.numpy as jnp
from jax import lax
from jax.experimental import pallas as pl
from jax.experimental.pallas import tpu as pltpu
from jax.experimental.pallas import tpu_sc as plsc

VOCAB = 23236
VOCABP = 23240
EMBED_DIM = 1728
SEG = 14
PADDED = SEG * 128
BATCH = 1024
SEQ = 50
SEQP = 56
TAB_ROWS = VOCAB * SEG
OUT_ROWS = BATCH * SEQ * SEG

NC = 2
NS = 16
NW = NC * NS
NSPLIT = 4
QROWS = OUT_ROWS // NSPLIT
RPW = QROWS // NW
CH = 112
NCHUNK = RPW // CH
SUB = 1
NIDX = CH // SUB

_MESH = plsc.VectorSubcoreMesh(core_axis_name="c", subcore_axis_name="s")


@functools.partial(
    pl.kernel,
    out_type=jax.ShapeDtypeStruct((QROWS, 128), jnp.float32),
    mesh=_MESH,
    compiler_params=pltpu.CompilerParams(use_tc_tiling_on_sc=False),
    scratch_types=[
        pltpu.VMEM((RPW,), jnp.int32),
        pltpu.VMEM((2, CH, 128), jnp.float32),
        pltpu.SemaphoreType.DMA,
        pltpu.SemaphoreType.DMA,
        pltpu.SemaphoreType.DMA,
    ],
)
def _glyph_gather(idx_hbm, tab_hbm, out_hbm, idx_v, rows_v, gsem, osem0, osem1):
    wid = lax.axis_index("s") * NC + lax.axis_index("c")
    base = wid * RPW
    osems = (osem0, osem1)

    pltpu.sync_copy(idx_hbm.at[pl.ds(wid * RPW, RPW)], idx_v)

    def start_gathers(j, b):
        for q in range(SUB):
            pltpu.async_copy(
                tab_hbm.at[idx_v.at[pl.ds(j * CH + q * NIDX, NIDX)]],
                rows_v.at[b, pl.ds(q * NIDX, NIDX)],
                gsem,
            )

    def wait_gathers(b):
        pltpu.make_async_copy(
            tab_hbm.at[pl.ds(0, CH)], rows_v.at[b], gsem
        ).wait()

    start_gathers(0, 0)

    def pair(p, carry):
        for b in range(2):
            j = 2 * p + b
            wait_gathers(b)

            @pl.when(j >= 1)
            def _():
                pltpu.make_async_copy(
                    rows_v.at[1 - b], out_hbm.at[pl.ds(base, CH)], osems[1 - b]
                ).wait()

            @pl.when(j + 1 < NCHUNK)
            def _():
                start_gathers(j + 1, 1 - b)

            pltpu.async_copy(
                rows_v.at[b], out_hbm.at[pl.ds(base + j * CH, CH)], osems[b]
            )
        return carry

    lax.fori_loop(0, NCHUNK // 2, pair, 0)
    pltpu.make_async_copy(
        rows_v.at[1], out_hbm.at[pl.ds(base, CH)], osem1
    ).wait()


_PAD_RB = 512


def _pad_body(t_ref, o_ref):
    x = t_ref[...]
    y = jnp.pad(x, ((0, 0), (0, PADDED - EMBED_DIM)))
    o_ref[...] = y.reshape(_PAD_RB * SEG, 128)


def _tc_pad(table):
    grid = (VOCAB + _PAD_RB - 1) // _PAD_RB
    return pl.pallas_call(
        _pad_body,
        grid=(grid,),
        in_specs=[pl.BlockSpec((_PAD_RB, EMBED_DIM), lambda i: (i, 0))],
        out_specs=pl.BlockSpec((_PAD_RB * SEG, 128), lambda i: (i, 0)),
        out_shape=jax.ShapeDtypeStruct((TAB_ROWS, 128), jnp.float32),
    )(table)


_DET_BB = 8
_QBATCH = BATCH // NSPLIT


def _det_body(x_ref, o_ref):
    x = x_ref[...]
    y = x.reshape(_DET_BB * SEQ, PADDED)[:, :EMBED_DIM]
    o_ref[...] = y.reshape(_DET_BB, SEQ, EMBED_DIM)


def _det_body_acc(x_ref, prev_ref, o_ref):
    del prev_ref
    _det_body(x_ref, o_ref)


def _tc_detile_q(x, quarter, prev):
    b0 = quarter * _QBATCH // _DET_BB
    out_spec = pl.BlockSpec(
        (_DET_BB, SEQ, EMBED_DIM), lambda b: (b + b0, 0, 0)
    )
    out_shape = jax.ShapeDtypeStruct((BATCH, SEQ, EMBED_DIM), jnp.float32)
    grid = (_QBATCH // _DET_BB,)
    in_spec = pl.BlockSpec((_DET_BB * SEQ * SEG, 128), lambda b: (b, 0))
    if prev is None:
        return pl.pallas_call(
            _det_body,
            grid=grid,
            in_specs=[in_spec],
            out_specs=out_spec,
            out_shape=out_shape,
        )(x)
    return pl.pallas_call(
        _det_body_acc,
        grid=grid,
        in_specs=[in_spec, pl.BlockSpec(memory_space=pl.ANY)],
        out_specs=out_spec,
        out_shape=out_shape,
        input_output_aliases={1: 0},
    )(x, prev)


def kernel(input_ids, embedding_table):
    ids = input_ids.reshape(-1).astype(jnp.int32)
    segs = (ids[:, None] * SEG + jnp.arange(SEG, dtype=jnp.int32)).reshape(-1)
    table_seg = _tc_pad(embedding_table)
    result = None
    for q in range(NSPLIT):
        xq = _glyph_gather(segs[q * QROWS:(q + 1) * QROWS], table_seg)
        result = _tc_detile_q(xq, q, result)
    return result


# speedup vs baseline: 5.0856x; 1.0049x over previous
import functools

import jax
import jax.numpy as jnp
from jax import lax
from jax.experimental import pallas as pl
from jax.experimental.pallas import tpu as pltpu
from jax.experimental.pallas import tpu_sc as plsc

VOCAB = 23236
VOCABP = 23240
EMBED_DIM = 1728
SEG = 14
PADDED = SEG * 128
BATCH = 1024
SEQ = 50
SEQP = 56
TAB_ROWS = VOCAB * SEG
OUT_ROWS = BATCH * SEQ * SEG

NC = 2
NS = 16
NW = NC * NS
NSPLIT = 2
QROWS = OUT_ROWS // NSPLIT
RPW = QROWS // NW
CH = 224
NCHUNK = RPW // CH
SUB = 2
NIDX = CH // SUB

_MESH = plsc.VectorSubcoreMesh(core_axis_name="c", subcore_axis_name="s")


@functools.partial(
    pl.kernel,
    out_type=jax.ShapeDtypeStruct((QROWS, 128), jnp.float32),
    mesh=_MESH,
    compiler_params=pltpu.CompilerParams(use_tc_tiling_on_sc=False),
    scratch_types=[
        pltpu.VMEM((RPW,), jnp.int32),
        pltpu.VMEM((2, CH, 128), jnp.float32),
        pltpu.SemaphoreType.DMA,
        pltpu.SemaphoreType.DMA,
        pltpu.SemaphoreType.DMA,
    ],
)
def _glyph_gather(idx_hbm, tab_hbm, out_hbm, idx_v, rows_v, gsem, osem0, osem1):
    wid = lax.axis_index("s") * NC + lax.axis_index("c")
    base = wid * RPW
    osems = (osem0, osem1)

    pltpu.sync_copy(idx_hbm.at[pl.ds(wid * RPW, RPW)], idx_v)

    def start_gathers(j, b):
        for q in range(SUB):
            pltpu.async_copy(
                tab_hbm.at[idx_v.at[pl.ds(j * CH + q * NIDX, NIDX)]],
                rows_v.at[b, pl.ds(q * NIDX, NIDX)],
                gsem,
            )

    def wait_gathers(b):
        pltpu.make_async_copy(
            tab_hbm.at[pl.ds(0, CH)], rows_v.at[b], gsem
        ).wait()

    start_gathers(0, 0)

    def pair(p, carry):
        for b in range(2):
            j = 2 * p + b
            wait_gathers(b)

            @pl.when(j >= 1)
            def _():
                pltpu.make_async_copy(
                    rows_v.at[1 - b], out_hbm.at[pl.ds(base, CH)], osems[1 - b]
                ).wait()

            @pl.when(j + 1 < NCHUNK)
            def _():
                start_gathers(j + 1, 1 - b)

            pltpu.async_copy(
                rows_v.at[b], out_hbm.at[pl.ds(base + j * CH, CH)], osems[b]
            )
        return carry

    lax.fori_loop(0, NCHUNK // 2, pair, 0)
    pltpu.make_async_copy(
        rows_v.at[1], out_hbm.at[pl.ds(base, CH)], osem1
    ).wait()


_PAD_RB = 512


def _pad_body(t_ref, o_ref):
    x = t_ref[...]
    y = jnp.pad(x, ((0, 0), (0, PADDED - EMBED_DIM)))
    o_ref[...] = y.reshape(_PAD_RB * SEG, 128)


def _tc_pad(table):
    grid = (VOCAB + _PAD_RB - 1) // _PAD_RB
    return pl.pallas_call(
        _pad_body,
        grid=(grid,),
        in_specs=[pl.BlockSpec((_PAD_RB, EMBED_DIM), lambda i: (i, 0))],
        out_specs=pl.BlockSpec((_PAD_RB * SEG, 128), lambda i: (i, 0)),
        out_shape=jax.ShapeDtypeStruct((TAB_ROWS, 128), jnp.float32),
    )(table)


_DET_BB = 8
_QBATCH = BATCH // NSPLIT


def _det_body(x_ref, o_ref):
    x = x_ref[...]
    y = x.reshape(_DET_BB * SEQ, PADDED)[:, :EMBED_DIM]
    o_ref[...] = y.reshape(_DET_BB, SEQ, EMBED_DIM)


def _det_body_acc(x_ref, prev_ref, o_ref):
    del prev_ref
    _det_body(x_ref, o_ref)


def _tc_detile_q(x, quarter, prev):
    b0 = quarter * _QBATCH // _DET_BB
    out_spec = pl.BlockSpec(
        (_DET_BB, SEQ, EMBED_DIM), lambda b: (b + b0, 0, 0)
    )
    out_shape = jax.ShapeDtypeStruct((BATCH, SEQ, EMBED_DIM), jnp.float32)
    grid = (_QBATCH // _DET_BB,)
    in_spec = pl.BlockSpec((_DET_BB * SEQ * SEG, 128), lambda b: (b, 0))
    if prev is None:
        return pl.pallas_call(
            _det_body,
            grid=grid,
            in_specs=[in_spec],
            out_specs=out_spec,
            out_shape=out_shape,
        )(x)
    return pl.pallas_call(
        _det_body_acc,
        grid=grid,
        in_specs=[in_spec, pl.BlockSpec(memory_space=pl.ANY)],
        out_specs=out_spec,
        out_shape=out_shape,
        input_output_aliases={1: 0},
    )(x, prev)


def kernel(input_ids, embedding_table):
    ids = input_ids.reshape(-1).astype(jnp.int32)
    segs = (ids[:, None] * SEG + jnp.arange(SEG, dtype=jnp.int32)).reshape(-1)
    table_seg = _tc_pad(embedding_table)
    result = None
    for q in range(NSPLIT):
        xq = _glyph_gather(segs[q * QROWS:(q + 1) * QROWS], table_seg)
        result = _tc_detile_q(xq, q, result)
    return result
